# baseline (speedup 1.0000x reference)
import functools

import jax
import jax.numpy as jnp
from jax import lax
from jax.experimental import pallas as pl
from jax.experimental.pallas import tpu as pltpu
from jax.experimental.pallas import tpu_sc as plsc

_N = 10000
_M = 320000
_D = 128

_NC = 2
_NS = 16
_NW = _NC * _NS
_MW = _M // _NW


@functools.cache
def _mesh():
    return plsc.VectorSubcoreMesh(core_axis_name="c", subcore_axis_name="s",
                                  num_cores=_NC, num_subcores=_NS)


_BG = 80
_NCG = _MW // _BG
_VPC = _BG // 16


def _count_chunk(cnt_vmem, idx_ref):
    for j in range(_VPC):
        v = idx_ref[pl.ds(j * 16, 16)]
        cnts, last = plsc.scan_count(v)
        plsc.addupdate_scatter(cnt_vmem, [v], cnts.astype(jnp.float32),
                               mask=last)


def _gather_body(mw, ncg,
                 h_hbm, heads_hbm, tails_hbm,
                 hh_out, ht_out, cnt_out,
                 hiA, tiA, hiB, tiB, hrA, trA, hrB, trB, cnt_vmem,
                 ihA, itA, g1A, g2A,
                 ihB, itB, g1B, g2B):
    c = lax.axis_index("c")
    s = lax.axis_index("s")
    wid = s * _NC + c
    base0 = wid * mw

    def czero(i, carry):
        cnt_vmem[pl.ds(i * 16, 16)] = jnp.zeros((16,), jnp.float32)
        return carry

    lax.fori_loop(0, _N // 16, czero, 0)

    def fire_idx(i, hi, ti, ih, it):
        base = base0 + i * _BG
        pltpu.async_copy(heads_hbm.at[pl.ds(base, _BG)], hi, ih)
        pltpu.async_copy(tails_hbm.at[pl.ds(base, _BG)], ti, it)

    def fire_gather(i, hi, ti, hr, tr, ih, it, g1, g2):
        base = base0 + i * _BG
        pltpu.make_async_copy(heads_hbm.at[pl.ds(base, _BG)], hi, ih).wait()
        pltpu.make_async_copy(tails_hbm.at[pl.ds(base, _BG)], ti, it).wait()
        pltpu.async_copy(h_hbm.at[hi], hr, g1)
        pltpu.async_copy(h_hbm.at[ti], tr, g2)
        _count_chunk(cnt_vmem, hi)
        _count_chunk(cnt_vmem, ti)

    def drain(i, hi, ti, hr, tr, g1, g2):
        base = base0 + i * _BG
        pltpu.make_async_copy(h_hbm.at[hi], hr, g1).wait()
        pltpu.make_async_copy(h_hbm.at[ti], tr, g2).wait()
        pltpu.sync_copy(hr, hh_out.at[pl.ds(base, _BG)])
        pltpu.sync_copy(tr, ht_out.at[pl.ds(base, _BG)])

    fire_idx(0, hiA, tiA, ihA, itA)
    fire_idx(1, hiB, tiB, ihB, itB)
    fire_gather(0, hiA, tiA, hrA, trA, ihA, itA, g1A, g2A)
    fire_gather(1, hiB, tiB, hrB, trB, ihB, itB, g1B, g2B)

    def step(k, carry):
        i = 2 * k
        drain(i, hiA, tiA, hrA, trA, g1A, g2A)

        @pl.when(i + 2 < ncg)
        def _():
            fire_idx(i + 2, hiA, tiA, ihA, itA)

        @pl.when(i + 1 < ncg)
        def _():
            drain(i + 1, hiB, tiB, hrB, trB, g1B, g2B)

        @pl.when(i + 3 < ncg)
        def _():
            fire_idx(i + 3, hiB, tiB, ihB, itB)

        @pl.when(i + 2 < ncg)
        def _():
            fire_gather(i + 2, hiA, tiA, hrA, trA, ihA, itA, g1A, g2A)

        @pl.when(i + 3 < ncg)
        def _():
            fire_gather(i + 3, hiB, tiB, hrB, trB, ihB, itB, g1B, g2B)

        return carry

    lax.fori_loop(0, (ncg + 1) // 2, step, 0)

    pltpu.sync_copy(cnt_vmem, cnt_out.at[wid, 0])


@functools.cache
def _gather(m):
    mw = m // _NW
    return pl.kernel(
        functools.partial(_gather_body, mw, mw // _BG),
        out_type=(
            jax.ShapeDtypeStruct((m, _D), jnp.float32),
            jax.ShapeDtypeStruct((m, _D), jnp.float32),
            jax.ShapeDtypeStruct((_NW, 1, _N), jnp.float32),
        ),
        mesh=_mesh(),
        scratch_types=[
            pltpu.VMEM((_BG,), jnp.int32),
            pltpu.VMEM((_BG,), jnp.int32),
            pltpu.VMEM((_BG,), jnp.int32),
            pltpu.VMEM((_BG,), jnp.int32),
            pltpu.VMEM((_BG, _D), jnp.float32),
            pltpu.VMEM((_BG, _D), jnp.float32),
            pltpu.VMEM((_BG, _D), jnp.float32),
            pltpu.VMEM((_BG, _D), jnp.float32),
            pltpu.VMEM((_N,), jnp.float32),
        ] + [pltpu.SemaphoreType.DMA] * 8,
        compiler_params=pltpu.CompilerParams(needs_layout_passes=False),
    )


def _msg_body(hh_ref, ht_ref, e_ref, r_ref, wf_ref, bf_ref, wb_ref, bb_ref,
              mf_ref, mb_ref):
    hh = hh_ref[...]
    ht = ht_ref[...]
    e = e_ref[...]
    r = r_ref[...]
    wf = wf_ref[...]
    wb = wb_ref[...]
    xf = jnp.concatenate([hh, e, hh * r, e * r], axis=1)
    xb = jnp.concatenate([ht, e, ht * r, e * r], axis=1)
    dn = (((1,), (1,)), ((), ()))
    mf_ref[...] = lax.dot_general(xf, wf, dn,
                                  preferred_element_type=jnp.float32) + bf_ref[...]
    mb_ref[...] = lax.dot_general(xb, wb, dn,
                                  preferred_element_type=jnp.float32) + bb_ref[...]


_BE = 2560


def _messages(hh, htl, e, r, wf, bf, wb, bb):
    m = hh.shape[0]
    grid = m // _BE
    edge_spec = pl.BlockSpec((_BE, _D), lambda i: (i, 0))
    w_spec = pl.BlockSpec((_D, 4 * _D), lambda i: (0, 0))
    b_spec = pl.BlockSpec((1, _D), lambda i: (0, 0))
    return pl.pallas_call(
        _msg_body,
        grid=(grid,),
        in_specs=[edge_spec, edge_spec, edge_spec, edge_spec,
                  w_spec, b_spec, w_spec, b_spec],
        out_specs=[edge_spec, edge_spec],
        out_shape=[
            jax.ShapeDtypeStruct((m, _D), jnp.float32),
            jax.ShapeDtypeStruct((m, _D), jnp.float32),
        ],
    )(hh, htl, e, r, wf, bf.reshape(1, _D), wb, bb.reshape(1, _D))


_BS = 80
_SZC = _N // _BS
_SZI = (_SZC + _NS - 1) // _NS


def _scatter_body(mw, ncs,
                  mf_hbm, mb_hbm, heads_hbm, tails_hbm, za_hbm,
                  agg_out,
                  hiA, tiA, hiB, tiB, mfA, mbA, mfB, mbB, agg_sh,
                  ihA, itA, lfA, lbA, sfA, sbA,
                  ihB, itB, lfB, lbB, sfB, sbB):
    c = lax.axis_index("c")
    s = lax.axis_index("s")

    pltpu.sync_copy(za_hbm, mfA)

    def zstep(k, carry):
        chunk = k * _NS + s

        @pl.when(chunk < _SZC)
        def _():
            pltpu.sync_copy(mfA, agg_sh.at[pl.ds(chunk * _BS, _BS)])

        return carry

    lax.fori_loop(0, _SZI, zstep, 0)
    plsc.subcore_barrier()

    base0 = (c * _NS + s) * mw

    def fire_pre(i, hi, ti, mf, mb, ih, it, lf, lb):
        base = base0 + i * _BS
        pltpu.async_copy(heads_hbm.at[pl.ds(base, _BS)], hi, ih)
        pltpu.async_copy(tails_hbm.at[pl.ds(base, _BS)], ti, it)
        pltpu.async_copy(mf_hbm.at[pl.ds(base, _BS)], mf, lf)
        pltpu.async_copy(mb_hbm.at[pl.ds(base, _BS)], mb, lb)

    def fire_scatter(i, hi, ti, mf, mb, ih, it, lf, lb, sf, sb):
        base = base0 + i * _BS
        pltpu.make_async_copy(heads_hbm.at[pl.ds(base, _BS)], hi, ih).wait()
        pltpu.make_async_copy(tails_hbm.at[pl.ds(base, _BS)], ti, it).wait()
        pltpu.make_async_copy(mf_hbm.at[pl.ds(base, _BS)], mf, lf).wait()
        pltpu.make_async_copy(mb_hbm.at[pl.ds(base, _BS)], mb, lb).wait()
        pltpu.async_copy(mf, agg_sh.at[ti], sf, add=True)
        pltpu.async_copy(mb, agg_sh.at[hi], sb, add=True)

    def drain(i, hi, ti, mf, mb, sf, sb):
        pltpu.make_async_copy(mf, agg_sh.at[ti], sf).wait()
        pltpu.make_async_copy(mb, agg_sh.at[hi], sb).wait()

    fire_pre(0, hiA, tiA, mfA, mbA, ihA, itA, lfA, lbA)
    fire_pre(1, hiB, tiB, mfB, mbB, ihB, itB, lfB, lbB)
    fire_scatter(0, hiA, tiA, mfA, mbA, ihA, itA, lfA, lbA, sfA, sbA)
    fire_scatter(1, hiB, tiB, mfB, mbB, ihB, itB, lfB, lbB, sfB, sbB)

    def step(k, carry):
        i = 2 * k
        drain(i, hiA, tiA, mfA, mbA, sfA, sbA)

        @pl.when(i + 2 < ncs)
        def _():
            fire_pre(i + 2, hiA, tiA, mfA, mbA, ihA, itA, lfA, lbA)

        @pl.when(i + 1 < ncs)
        def _():
            drain(i + 1, hiB, tiB, mfB, mbB, sfB, sbB)

        @pl.when(i + 3 < ncs)
        def _():
            fire_pre(i + 3, hiB, tiB, mfB, mbB, ihB, itB, lfB, lbB)

        @pl.when(i + 2 < ncs)
        def _():
            fire_scatter(i + 2, hiA, tiA, mfA, mbA, ihA, itA, lfA, lbA, sfA, sbA)

        @pl.when(i + 3 < ncs)
        def _():
            fire_scatter(i + 3, hiB, tiB, mfB, mbB, ihB, itB, lfB, lbB, sfB, sbB)

        return carry

    lax.fori_loop(0, (ncs + 1) // 2, step, 0)
    plsc.subcore_barrier()

    def wstep(k, carry):
        chunk = k * _NS + s

        @pl.when(chunk < _SZC)
        def _():
            r0 = chunk * _BS
            pltpu.sync_copy(agg_sh.at[pl.ds(r0, _BS)], mfA)
            pltpu.sync_copy(mfA, agg_out.at[c, pl.ds(r0, _BS)])

        return carry

    lax.fori_loop(0, _SZI, wstep, 0)


@functools.cache
def _scatter(m):
    mw = m // _NW
    return pl.kernel(
        functools.partial(_scatter_body, mw, mw // _BS),
        out_type=jax.ShapeDtypeStruct((_NC, _N, _D), jnp.float32),
        mesh=_mesh(),
        scratch_types=[
            pltpu.VMEM((_BS,), jnp.int32),
            pltpu.VMEM((_BS,), jnp.int32),
            pltpu.VMEM((_BS,), jnp.int32),
            pltpu.VMEM((_BS,), jnp.int32),
            pltpu.VMEM((_BS, _D), jnp.float32),
            pltpu.VMEM((_BS, _D), jnp.float32),
            pltpu.VMEM((_BS, _D), jnp.float32),
            pltpu.VMEM((_BS, _D), jnp.float32),
            pltpu.VMEM_SHARED((_N, _D), jnp.float32),
        ] + [pltpu.SemaphoreType.DMA] * 12,
    )


def _epi_body(agg1_ref, agg2_ref, cnt1_ref, cnt2_ref, h_ref, g_ref, b_ref,
              out_ref):
    agg = (agg1_ref[0] + agg1_ref[1]) + (agg2_ref[0] + agg2_ref[1])
    cnt = (jnp.sum(cnt1_ref[...], axis=1)
           + jnp.sum(cnt2_ref[...], axis=1))[:, None]
    x = agg / cnt
    x = jnp.where(x >= 0, x, 0.01 * x) + h_ref[...]
    mean = jnp.mean(x, axis=-1, keepdims=True)
    var = jnp.mean(jnp.square(x - mean), axis=-1, keepdims=True)
    out_ref[...] = (x - mean) * lax.rsqrt(var + 1e-5) * g_ref[...] + b_ref[...]


_BN = 2000


def _epilogue(agg1, agg2, cnt1, cnt2, h, g, b):
    grid = _N // _BN
    agg_spec = pl.BlockSpec((_NC, _BN, _D), lambda i: (0, i, 0))
    cnt_spec = pl.BlockSpec((_BN, _NW), lambda i: (i, 0))
    return pl.pallas_call(
        _epi_body,
        grid=(grid,),
        in_specs=[
            agg_spec,
            agg_spec,
            cnt_spec,
            cnt_spec,
            pl.BlockSpec((_BN, _D), lambda i: (i, 0)),
            pl.BlockSpec((1, _D), lambda i: (0, 0)),
            pl.BlockSpec((1, _D), lambda i: (0, 0)),
        ],
        out_specs=pl.BlockSpec((_BN, _D), lambda i: (i, 0)),
        out_shape=jax.ShapeDtypeStruct((_N, _D), jnp.float32),
    )(agg1, agg2, cnt1, cnt2, h, g.reshape(1, _D), b.reshape(1, _D))


_MS = 62 * _BE


def kernel(H, E, ht, r_embed, W_fwd, b_fwd, W_back, b_back, ln_gamma, ln_beta):
    heads = ht[:, 0]
    tails = ht[:, 1]
    za80 = jnp.zeros((_BS, _D), jnp.float32)

    def half(sl, m):
        hd, tl = heads[sl], tails[sl]
        hh, htl, cnt = _gather(m)(H, hd, tl)
        mf, mb = _messages(hh, htl, E[sl], r_embed[sl],
                           W_fwd, b_fwd, W_back, b_back)
        agg = _scatter(m)(mf, mb, hd, tl, za80)
        return agg, cnt.reshape(_NW, _N).T

    agg1, cnt1 = half(slice(0, _MS), _MS)
    agg2, cnt2 = half(slice(_MS, _M), _M - _MS)
    return _epilogue(agg1, agg2, cnt1, cnt2, H, ln_gamma, ln_beta)


# speedup vs baseline: 1.2149x; 1.2149x over previous
import functools

import jax
import jax.numpy as jnp
from jax import lax
from jax.experimental import pallas as pl
from jax.experimental.pallas import tpu as pltpu
from jax.experimental.pallas import tpu_sc as plsc

_N = 10000
_M = 320000
_D = 128

_NC = 2
_NS = 16
_NW = _NC * _NS
_MW = _M // _NW


@functools.cache
def _mesh():
    return plsc.VectorSubcoreMesh(core_axis_name="c", subcore_axis_name="s",
                                  num_cores=_NC, num_subcores=_NS)


_BG = 80
_NCG = _MW // _BG
_VPC = _BG // 16


def _count_chunk(cnt_vmem, idx_ref, o):
    for j in range(_VPC):
        v = idx_ref[pl.ds(o + j * 16, 16)]
        cnts, last = plsc.scan_count(v)
        plsc.addupdate_scatter(cnt_vmem, [v], cnts.astype(jnp.float32),
                               mask=last)


def _gather_body(mw, ncg,
                 h_hbm, heads_hbm, tails_hbm,
                 hh_out, ht_out, cnt_out,
                 hiall, tiall, hrA, trA, hrB, trB, cnt_vmem,
                 g1A, g2A, g1B, g2B):
    c = lax.axis_index("c")
    s = lax.axis_index("s")
    wid = s * _NC + c
    base0 = wid * mw

    pltpu.sync_copy(heads_hbm.at[pl.ds(base0, mw)], hiall)
    pltpu.sync_copy(tails_hbm.at[pl.ds(base0, mw)], tiall)

    def czero(i, carry):
        cnt_vmem[pl.ds(i * 16, 16)] = jnp.zeros((16,), jnp.float32)
        return carry

    lax.fori_loop(0, _N // 16, czero, 0)

    def fire(i, hr, tr, g1, g2):
        o = i * _BG
        pltpu.async_copy(h_hbm.at[hiall.at[pl.ds(o, _BG)]], hr, g1)
        pltpu.async_copy(h_hbm.at[tiall.at[pl.ds(o, _BG)]], tr, g2)
        _count_chunk(cnt_vmem, hiall, o)
        _count_chunk(cnt_vmem, tiall, o)

    def drain(i, hr, tr, g1, g2):
        o = i * _BG
        pltpu.make_async_copy(h_hbm.at[hiall.at[pl.ds(o, _BG)]], hr, g1).wait()
        pltpu.make_async_copy(h_hbm.at[tiall.at[pl.ds(o, _BG)]], tr, g2).wait()
        pltpu.sync_copy(hr, hh_out.at[pl.ds(base0 + o, _BG)])
        pltpu.sync_copy(tr, ht_out.at[pl.ds(base0 + o, _BG)])

    fire(0, hrA, trA, g1A, g2A)
    fire(1, hrB, trB, g1B, g2B)

    def step(k, carry):
        i = 2 * k
        drain(i, hrA, trA, g1A, g2A)

        @pl.when(i + 2 < ncg)
        def _():
            fire(i + 2, hrA, trA, g1A, g2A)

        @pl.when(i + 1 < ncg)
        def _():
            drain(i + 1, hrB, trB, g1B, g2B)

        @pl.when(i + 3 < ncg)
        def _():
            fire(i + 3, hrB, trB, g1B, g2B)

        return carry

    lax.fori_loop(0, (ncg + 1) // 2, step, 0)

    pltpu.sync_copy(cnt_vmem, cnt_out.at[wid, 0])


@functools.cache
def _gather(m):
    mw = m // _NW
    return pl.kernel(
        functools.partial(_gather_body, mw, mw // _BG),
        out_type=(
            jax.ShapeDtypeStruct((m, _D), jnp.float32),
            jax.ShapeDtypeStruct((m, _D), jnp.float32),
            jax.ShapeDtypeStruct((_NW, 1, _N), jnp.float32),
        ),
        mesh=_mesh(),
        scratch_types=[
            pltpu.VMEM((mw,), jnp.int32),
            pltpu.VMEM((mw,), jnp.int32),
            pltpu.VMEM((_BG, _D), jnp.float32),
            pltpu.VMEM((_BG, _D), jnp.float32),
            pltpu.VMEM((_BG, _D), jnp.float32),
            pltpu.VMEM((_BG, _D), jnp.float32),
            pltpu.VMEM((_N,), jnp.float32),
        ] + [pltpu.SemaphoreType.DMA] * 4,
        compiler_params=pltpu.CompilerParams(needs_layout_passes=False),
    )


def _msg_body(hh_ref, ht_ref, e_ref, r_ref, wf_ref, bf_ref, wb_ref, bb_ref,
              mf_ref, mb_ref):
    hh = hh_ref[...]
    ht = ht_ref[...]
    e = e_ref[...]
    r = r_ref[...]
    wf = wf_ref[...]
    wb = wb_ref[...]
    xf = jnp.concatenate([hh, e, hh * r, e * r], axis=1)
    xb = jnp.concatenate([ht, e, ht * r, e * r], axis=1)
    dn = (((1,), (1,)), ((), ()))
    mf_ref[...] = lax.dot_general(xf, wf, dn,
                                  preferred_element_type=jnp.float32) + bf_ref[...]
    mb_ref[...] = lax.dot_general(xb, wb, dn,
                                  preferred_element_type=jnp.float32) + bb_ref[...]


_BE = 4000


def _messages(hh, htl, e, r, wf, bf, wb, bb):
    m = hh.shape[0]
    grid = m // _BE
    edge_spec = pl.BlockSpec((_BE, _D), lambda i: (i, 0))
    w_spec = pl.BlockSpec((_D, 4 * _D), lambda i: (0, 0))
    b_spec = pl.BlockSpec((1, _D), lambda i: (0, 0))
    return pl.pallas_call(
        _msg_body,
        grid=(grid,),
        in_specs=[edge_spec, edge_spec, edge_spec, edge_spec,
                  w_spec, b_spec, w_spec, b_spec],
        out_specs=[edge_spec, edge_spec],
        out_shape=[
            jax.ShapeDtypeStruct((m, _D), jnp.float32),
            jax.ShapeDtypeStruct((m, _D), jnp.float32),
        ],
    )(hh, htl, e, r, wf, bf.reshape(1, _D), wb, bb.reshape(1, _D))


_BS = 80
_SZC = _N // _BS
_SZI = (_SZC + _NS - 1) // _NS


def _scatter_body(mw, ncs,
                  mf_hbm, mb_hbm, heads_hbm, tails_hbm, za_hbm,
                  agg_out,
                  hiA, tiA, hiB, tiB, mfA, mbA, mfB, mbB, agg_sh,
                  ihA, itA, lfA, lbA, sfA, sbA,
                  ihB, itB, lfB, lbB, sfB, sbB):
    c = lax.axis_index("c")
    s = lax.axis_index("s")

    pltpu.sync_copy(za_hbm, mfA)

    def zstep(k, carry):
        chunk = k * _NS + s

        @pl.when(chunk < _SZC)
        def _():
            pltpu.sync_copy(mfA, agg_sh.at[pl.ds(chunk * _BS, _BS)])

        return carry

    lax.fori_loop(0, _SZI, zstep, 0)
    plsc.subcore_barrier()

    base0 = (c * _NS + s) * mw

    def fire_pre(i, hi, ti, mf, mb, ih, it, lf, lb):
        base = base0 + i * _BS
        pltpu.async_copy(heads_hbm.at[pl.ds(base, _BS)], hi, ih)
        pltpu.async_copy(tails_hbm.at[pl.ds(base, _BS)], ti, it)
        pltpu.async_copy(mf_hbm.at[pl.ds(base, _BS)], mf, lf)
        pltpu.async_copy(mb_hbm.at[pl.ds(base, _BS)], mb, lb)

    def fire_scatter(i, hi, ti, mf, mb, ih, it, lf, lb, sf, sb):
        base = base0 + i * _BS
        pltpu.make_async_copy(heads_hbm.at[pl.ds(base, _BS)], hi, ih).wait()
        pltpu.make_async_copy(tails_hbm.at[pl.ds(base, _BS)], ti, it).wait()
        pltpu.make_async_copy(mf_hbm.at[pl.ds(base, _BS)], mf, lf).wait()
        pltpu.make_async_copy(mb_hbm.at[pl.ds(base, _BS)], mb, lb).wait()
        pltpu.async_copy(mf, agg_sh.at[ti], sf, add=True)
        pltpu.async_copy(mb, agg_sh.at[hi], sb, add=True)

    def drain(i, hi, ti, mf, mb, sf, sb):
        pltpu.make_async_copy(mf, agg_sh.at[ti], sf).wait()
        pltpu.make_async_copy(mb, agg_sh.at[hi], sb).wait()

    fire_pre(0, hiA, tiA, mfA, mbA, ihA, itA, lfA, lbA)
    fire_pre(1, hiB, tiB, mfB, mbB, ihB, itB, lfB, lbB)
    fire_scatter(0, hiA, tiA, mfA, mbA, ihA, itA, lfA, lbA, sfA, sbA)
    fire_scatter(1, hiB, tiB, mfB, mbB, ihB, itB, lfB, lbB, sfB, sbB)

    def step(k, carry):
        i = 2 * k
        drain(i, hiA, tiA, mfA, mbA, sfA, sbA)

        @pl.when(i + 2 < ncs)
        def _():
            fire_pre(i + 2, hiA, tiA, mfA, mbA, ihA, itA, lfA, lbA)

        @pl.when(i + 1 < ncs)
        def _():
            drain(i + 1, hiB, tiB, mfB, mbB, sfB, sbB)

        @pl.when(i + 3 < ncs)
        def _():
            fire_pre(i + 3, hiB, tiB, mfB, mbB, ihB, itB, lfB, lbB)

        @pl.when(i + 2 < ncs)
        def _():
            fire_scatter(i + 2, hiA, tiA, mfA, mbA, ihA, itA, lfA, lbA, sfA, sbA)

        @pl.when(i + 3 < ncs)
        def _():
            fire_scatter(i + 3, hiB, tiB, mfB, mbB, ihB, itB, lfB, lbB, sfB, sbB)

        return carry

    lax.fori_loop(0, (ncs + 1) // 2, step, 0)
    plsc.subcore_barrier()

    def wstep(k, carry):
        chunk = k * _NS + s

        @pl.when(chunk < _SZC)
        def _():
            r0 = chunk * _BS
            pltpu.sync_copy(agg_sh.at[pl.ds(r0, _BS)], mfA)
            pltpu.sync_copy(mfA, agg_out.at[c, pl.ds(r0, _BS)])

        return carry

    lax.fori_loop(0, _SZI, wstep, 0)


@functools.cache
def _scatter(m):
    mw = m // _NW
    return pl.kernel(
        functools.partial(_scatter_body, mw, mw // _BS),
        out_type=jax.ShapeDtypeStruct((_NC, _N, _D), jnp.float32),
        mesh=_mesh(),
        scratch_types=[
            pltpu.VMEM((_BS,), jnp.int32),
            pltpu.VMEM((_BS,), jnp.int32),
            pltpu.VMEM((_BS,), jnp.int32),
            pltpu.VMEM((_BS,), jnp.int32),
            pltpu.VMEM((_BS, _D), jnp.float32),
            pltpu.VMEM((_BS, _D), jnp.float32),
            pltpu.VMEM((_BS, _D), jnp.float32),
            pltpu.VMEM((_BS, _D), jnp.float32),
            pltpu.VMEM_SHARED((_N, _D), jnp.float32),
        ] + [pltpu.SemaphoreType.DMA] * 12,
    )


def _epi_body(agg_ref, cnt_ref, h_ref, g_ref, b_ref, out_ref):
    agg = agg_ref[0] + agg_ref[1]
    cnt = jnp.sum(cnt_ref[...], axis=1)[:, None]
    x = agg / cnt
    x = jnp.where(x >= 0, x, 0.01 * x) + h_ref[...]
    mean = jnp.mean(x, axis=-1, keepdims=True)
    var = jnp.mean(jnp.square(x - mean), axis=-1, keepdims=True)
    out_ref[...] = (x - mean) * lax.rsqrt(var + 1e-5) * g_ref[...] + b_ref[...]


_BN = 2000


def _epilogue(agg_parts, cnt_parts, h, g, b):
    grid = _N // _BN
    return pl.pallas_call(
        _epi_body,
        grid=(grid,),
        in_specs=[
            pl.BlockSpec((_NC, _BN, _D), lambda i: (0, i, 0)),
            pl.BlockSpec((_BN, _NW), lambda i: (i, 0)),
            pl.BlockSpec((_BN, _D), lambda i: (i, 0)),
            pl.BlockSpec((1, _D), lambda i: (0, 0)),
            pl.BlockSpec((1, _D), lambda i: (0, 0)),
        ],
        out_specs=pl.BlockSpec((_BN, _D), lambda i: (i, 0)),
        out_shape=jax.ShapeDtypeStruct((_N, _D), jnp.float32),
    )(agg_parts, cnt_parts, h, g.reshape(1, _D), b.reshape(1, _D))


def kernel(H, E, ht, r_embed, W_fwd, b_fwd, W_back, b_back, ln_gamma, ln_beta):
    heads = ht[:, 0]
    tails = ht[:, 1]
    za80 = jnp.zeros((_BS, _D), jnp.float32)
    hh, htl, cnt = _gather(_M)(H, heads, tails)
    mf, mb = _messages(hh, htl, E, r_embed, W_fwd, b_fwd, W_back, b_back)
    agg = _scatter(_M)(mf, mb, heads, tails, za80)
    return _epilogue(agg, cnt.reshape(_NW, _N).T, H, ln_gamma, ln_beta)
